# baseline (speedup 1.0000x reference)
import functools

import jax
import jax.numpy as jnp
from jax import lax
from jax.experimental import pallas as pl
from jax.experimental.pallas import tpu as pltpu
from jax.experimental.pallas import tpu_sc as plsc

_B, _C, _H, _W = 8, 80, 128, 128
_K = 64
_CBLK = 16
_NCHUNK = _C // _CBLK
_BIG = 2**30


_BG = 1


def _topk_body(hm_ref, vals_ref, pix_ref, cls_ref, nms_scr, sum_scr, lvl2_scr):
    cc = pl.program_id(1)
    s = jax.nn.sigmoid(hm_ref[...])
    zw = jnp.zeros((_BG, _CBLK, _H, 1), jnp.float32)
    m3 = jnp.maximum(s, jnp.maximum(
        jnp.concatenate([zw, s[:, :, :, :-1]], axis=3),
        jnp.concatenate([s[:, :, :, 1:], zw], axis=3)))
    zh = jnp.zeros((_BG, _CBLK, 1, _W), jnp.float32)
    hmax = jnp.maximum(m3, jnp.maximum(
        jnp.concatenate([zh, m3[:, :, :-1, :]], axis=2),
        jnp.concatenate([m3[:, :, 1:, :], zh], axis=2)))
    nms = jnp.where(s == hmax, s, 0.0)
    nms_scr[:, pl.ds(cc * _CBLK, _CBLK)] = nms
    sum_scr[:, pl.ds(cc * _CBLK, _CBLK)] = jnp.max(nms, axis=3)

    @pl.when(cc == _NCHUNK - 1)
    def _extract():
        w_iota = lax.broadcasted_iota(jnp.int32, (1, 1, _W), 2)
        h_iota = lax.broadcasted_iota(jnp.int32, (1, _H), 1)
        c_iota = lax.broadcasted_iota(jnp.int32, (1, _W), 1)

        for g in range(_BG):
            l2r = jnp.max(jnp.transpose(sum_scr[g]), axis=0, keepdims=True)
            lvl2_scr[g] = jnp.concatenate(
                [l2r, jnp.full((1, _W - _C), -1.0, jnp.float32)], axis=1)

        def body(k, carry):
            for g in range(_BG):
                l2 = lvl2_scr[g]
                m = jnp.max(l2)
                c = jnp.min(jnp.where(l2 == m, c_iota, _BIG))
                srow = sum_scr[g, pl.ds(c, 1), :]
                h = jnp.min(jnp.where(srow == m, h_iota, _BIG))
                row = nms_scr[g, pl.ds(c, 1), pl.ds(h, 1), :]
                w = jnp.min(jnp.where(row == m, w_iota, _BIG))
                vals_ref[g, pl.ds(k, 1), :] = jnp.full((1, 1), m, jnp.float32)
                pix_ref[g, pl.ds(k, 1), :] = jnp.full((1, 1), h * _W + w,
                                                      jnp.int32)
                cls_ref[g, pl.ds(k, 1), :] = jnp.full((1, 1), c, jnp.int32)
                row2 = jnp.where(w_iota == w, 0.0, row)
                nms_scr[g, pl.ds(c, 1), pl.ds(h, 1), :] = row2
                srow2 = jnp.where(h_iota == h, jnp.max(row2), srow)
                sum_scr[g, pl.ds(c, 1), :] = srow2
                lvl2_scr[g] = jnp.where(c_iota == c, jnp.max(srow2), l2)
            return carry

        lax.fori_loop(0, _K, body, 0)


def _run_topk(hm, interpret=False):
    return pl.pallas_call(
        _topk_body,
        grid=(_B // _BG, _NCHUNK),
        in_specs=[pl.BlockSpec((_BG, _CBLK, _H, _W),
                               lambda g, cc: (g, cc, 0, 0))],
        out_specs=[
            pl.BlockSpec((_BG, _K, 1), lambda g, cc: (g, 0, 0)),
            pl.BlockSpec((_BG, _K, 1), lambda g, cc: (g, 0, 0)),
            pl.BlockSpec((_BG, _K, 1), lambda g, cc: (g, 0, 0)),
        ],
        out_shape=[
            jax.ShapeDtypeStruct((_B, _K, 1), jnp.float32),
            jax.ShapeDtypeStruct((_B, _K, 1), jnp.int32),
            jax.ShapeDtypeStruct((_B, _K, 1), jnp.int32),
        ],
        scratch_shapes=[
            pltpu.VMEM((_BG, _C, _H, _W), jnp.float32),
            pltpu.VMEM((_BG, _C, _H), jnp.float32),
            pltpu.VMEM((_BG, 1, _W), jnp.float32),
        ],
        interpret=interpret,
    )(hm)


_NCV = 176
_CWROWS = 169 * 128


def _sc_gather_body(reg_t, wh_t, cw_t, pixf, valsf, clsf,
                    bbox_out, convw_out,
                    pix_v, vals_v, cls_v, ridx0_v, ridx1_v, idx_v, rbuf,
                    rg_rows, conv_res, bbox_v, sem_rw, s0, s1, s2, s3):
    slot_sems = (s0, s1, s2, s3)
    nc = 2
    wid = lax.axis_index("s") * nc + lax.axis_index("c")
    b = wid // 4
    row0 = b * _K + (wid % 4) * 16

    pltpu.sync_copy(pixf.at[pl.ds(row0, 16)], pix_v)
    pltpu.sync_copy(valsf.at[pl.ds(row0, 16)], vals_v)
    pltpu.sync_copy(clsf.at[pl.ds(row0, 16)], cls_v)

    pix = pix_v[...]
    lane = lax.bitwise_and(pix, 127)
    wordrow = lax.shift_right_logical(pix, 7)
    iota = lax.iota(jnp.int32, 16)

    ridx0_v[...] = b * 256 + wordrow
    ridx1_v[...] = b * 256 + 128 + wordrow
    handles_rw = [
        pltpu.async_copy(reg_t.at[ridx0_v], rg_rows.at[0], sem_rw),
        pltpu.async_copy(reg_t.at[ridx1_v], rg_rows.at[1], sem_rw),
        pltpu.async_copy(wh_t.at[ridx0_v], rg_rows.at[2], sem_rw),
        pltpu.async_copy(wh_t.at[ridx1_v], rg_rows.at[3], sem_rw),
    ]

    for k in range(16):
        pk = jnp.max(jnp.where(iota == k, pix, -1))
        rowbase = b * _CWROWS + lax.shift_right_logical(pk, 7)
        for j2 in range(2):
            for t in range(6):
                ch = jnp.minimum(iota + (j2 * 96 + t * 16), 168)
                plsc.store_scatter(idx_v, [iota + (k * 192 + j2 * 96 + t * 16)],
                                   rowbase + ch * 128)

    def fire(pos):
        slot = pos % 4
        k, j2 = pos // 2, pos % 2
        return pltpu.async_copy(
            cw_t.at[idx_v.at[pl.ds(k * 192 + j2 * 96, 96)]],
            rbuf.at[slot], slot_sems[slot])

    handles = {}
    for pos in range(4):
        handles[pos] = fire(pos)
    for pos in range(32):
        k, j2 = pos // 2, pos % 2
        slot = pos % 4
        handles.pop(pos).wait()
        pk = jnp.max(jnp.where(iota == k, pix, -1))
        lk = jnp.full((16,), lax.bitwise_and(pk, 127), jnp.int32)
        for jj in range(6 if j2 == 0 else 5):
            cbase = j2 * 96 + jj * 16
            vals = plsc.load_gather(rbuf.at[slot], [iota + jj * 16, lk])
            plsc.store_scatter(conv_res, [iota + (k * _NCV + cbase)], vals)
        if pos + 4 < 32:
            handles[pos + 4] = fire(pos + 4)

    for h in handles_rw:
        h.wait()
    r0 = plsc.load_gather(rg_rows, [jnp.full((16,), 0, jnp.int32), iota, lane])
    r1 = plsc.load_gather(rg_rows, [jnp.full((16,), 1, jnp.int32), iota, lane])
    w0 = plsc.load_gather(rg_rows, [jnp.full((16,), 2, jnp.int32), iota, lane])
    w1 = plsc.load_gather(rg_rows, [jnp.full((16,), 3, jnp.int32), iota, lane])

    xs = lax.bitwise_and(pix, _W - 1).astype(jnp.float32)
    ys = lax.shift_right_logical(pix, 7).astype(jnp.float32)
    cx = xs + r0
    cy = ys + r1
    cols = [cx - w0 / 2, cy - w1 / 2, cx + w0 / 2, cy + w1 / 2,
            vals_v[...], cls_v[...].astype(jnp.float32),
            jnp.zeros((16,), jnp.float32), jnp.zeros((16,), jnp.float32)]
    for j, colv in enumerate(cols):
        plsc.store_scatter(bbox_v, [iota * 8 + j], colv)

    pltpu.sync_copy(bbox_v, bbox_out.at[pl.ds(row0 * 8, 128)])
    pltpu.sync_copy(conv_res, convw_out.at[pl.ds(row0 * _NCV, 16 * _NCV)])


def _run_sc_gather(reg_t, wh_t, cw_t, pixf, valsf, clsf):
    return pl.kernel(
        _sc_gather_body,
        mesh=plsc.VectorSubcoreMesh(core_axis_name="c", subcore_axis_name="s"),
        compiler_params=pltpu.CompilerParams(needs_layout_passes=False),
        out_type=[
            jax.ShapeDtypeStruct((_B * _K * 8,), jnp.float32),
            jax.ShapeDtypeStruct((_B * _K * _NCV,), jnp.float32),
        ],
        scratch_types=[
            pltpu.VMEM((16,), jnp.int32),
            pltpu.VMEM((16,), jnp.float32),
            pltpu.VMEM((16,), jnp.int32),
            pltpu.VMEM((16,), jnp.int32),
            pltpu.VMEM((16,), jnp.int32),
            pltpu.VMEM((16 * 192,), jnp.int32),
            pltpu.VMEM((4, 96, 128), jnp.float32),
            pltpu.VMEM((4, 16, 128), jnp.float32),
            pltpu.VMEM((16 * _NCV,), jnp.float32),
            pltpu.VMEM((128,), jnp.float32),
            pltpu.SemaphoreType.DMA,
            pltpu.SemaphoreType.DMA,
            pltpu.SemaphoreType.DMA,
            pltpu.SemaphoreType.DMA,
            pltpu.SemaphoreType.DMA,
        ],
    )(reg_t, wh_t, cw_t, pixf, valsf, clsf)


def kernel(hm, reg, wh, seg_feat, conv_weight):
    vals3, pix3, cls3 = _run_topk(hm)
    bbox_flat, convw = _run_sc_gather(
        reg.reshape(_B * 2 * _H, _W),
        wh.reshape(_B * 2 * _H, _W),
        conv_weight.reshape(_B * 169 * _H, _W),
        pix3.reshape(_B * _K),
        vals3.reshape(_B * _K),
        cls3.reshape(_B * _K),
    )
    bboxes = bbox_flat.reshape(_B, _K, 8)[:, :, :6]
    conv_g = convw.reshape(_B, _K, _NCV)[:, :, :169]
    return (bboxes, seg_feat, conv_g)


# speedup vs baseline: 1.3704x; 1.3704x over previous
import functools

import jax
import jax.numpy as jnp
from jax import lax
from jax.experimental import pallas as pl
from jax.experimental.pallas import tpu as pltpu
from jax.experimental.pallas import tpu_sc as plsc

_B, _C, _H, _W = 8, 80, 128, 128
_K = 64
_CBLK = 16
_NCHUNK = _C // _CBLK
_BIG = 2**30


_BG = 4


def _topk_body(hm_ref, vals_ref, pix_ref, cls_ref, nms_scr, sum_scr, lvl2_scr):
    cc = pl.program_id(1)
    s = jax.nn.sigmoid(hm_ref[...])
    zw = jnp.zeros((_BG, _CBLK, _H, 1), jnp.float32)
    m3 = jnp.maximum(s, jnp.maximum(
        jnp.concatenate([zw, s[:, :, :, :-1]], axis=3),
        jnp.concatenate([s[:, :, :, 1:], zw], axis=3)))
    zh = jnp.zeros((_BG, _CBLK, 1, _W), jnp.float32)
    hmax = jnp.maximum(m3, jnp.maximum(
        jnp.concatenate([zh, m3[:, :, :-1, :]], axis=2),
        jnp.concatenate([m3[:, :, 1:, :], zh], axis=2)))
    nms = jnp.where(s == hmax, s, 0.0)
    nms_scr[:, pl.ds(cc * _CBLK, _CBLK)] = nms
    sum_scr[:, pl.ds(cc * _CBLK, _CBLK)] = jnp.max(nms, axis=3)

    @pl.when(cc == _NCHUNK - 1)
    def _extract():
        ch_iota = (lax.broadcasted_iota(jnp.int32, (_C, _H), 0) * _H
                   + lax.broadcasted_iota(jnp.int32, (_C, _H), 1))
        w_iota = lax.broadcasted_iota(jnp.int32, (1, 1, _W), 2)
        h_iota = lax.broadcasted_iota(jnp.int32, (1, _H), 1)

        def body(k, carry):
            for g in range(_BG):
                summ = sum_scr[g]
                m = jnp.max(summ)
                f = jnp.min(jnp.where(summ == m, ch_iota, _BIG))
                c = f // _H
                h = f - c * _H
                row = nms_scr[g, pl.ds(c, 1), pl.ds(h, 1), :]
                w = jnp.min(jnp.where(row == m, w_iota, _BIG))
                vals_ref[g, pl.ds(k, 1), :] = jnp.full((1, 1), m, jnp.float32)
                pix_ref[g, pl.ds(k, 1), :] = jnp.full((1, 1), h * _W + w,
                                                      jnp.int32)
                cls_ref[g, pl.ds(k, 1), :] = jnp.full((1, 1), c, jnp.int32)
                row2 = jnp.where(w_iota == w, 0.0, row)
                nms_scr[g, pl.ds(c, 1), pl.ds(h, 1), :] = row2
                srow = sum_scr[g, pl.ds(c, 1), :]
                sum_scr[g, pl.ds(c, 1), :] = jnp.where(h_iota == h,
                                                       jnp.max(row2), srow)
            return carry

        lax.fori_loop(0, _K, body, 0)


def _run_topk(hm, interpret=False):
    return pl.pallas_call(
        _topk_body,
        grid=(_B // _BG, _NCHUNK),
        in_specs=[pl.BlockSpec((_BG, _CBLK, _H, _W),
                               lambda g, cc: (g, cc, 0, 0))],
        out_specs=[
            pl.BlockSpec((_BG, _K, 1), lambda g, cc: (g, 0, 0)),
            pl.BlockSpec((_BG, _K, 1), lambda g, cc: (g, 0, 0)),
            pl.BlockSpec((_BG, _K, 1), lambda g, cc: (g, 0, 0)),
        ],
        out_shape=[
            jax.ShapeDtypeStruct((_B, _K, 1), jnp.float32),
            jax.ShapeDtypeStruct((_B, _K, 1), jnp.int32),
            jax.ShapeDtypeStruct((_B, _K, 1), jnp.int32),
        ],
        scratch_shapes=[
            pltpu.VMEM((_BG, _C, _H, _W), jnp.float32),
            pltpu.VMEM((_BG, _C, _H), jnp.float32),
            pltpu.VMEM((_BG, 1, _W), jnp.float32),
        ],
        interpret=interpret,
    )(hm)


_NCV = 176
_CWROWS = 169 * 128


def _sc_gather_body(reg_t, wh_t, cw_t, pixf, valsf, clsf,
                    bbox_out, convw_out,
                    pix_v, vals_v, cls_v, ridx0_v, ridx1_v, idx_v, rbuf,
                    rg_rows, conv_res, bbox_v, sem_rw, s0, s1, s2, s3):
    slot_sems = (s0, s1, s2, s3)
    nc = 2
    wid = lax.axis_index("s") * nc + lax.axis_index("c")
    b = wid // 4
    row0 = b * _K + (wid % 4) * 16

    pltpu.sync_copy(pixf.at[pl.ds(row0, 16)], pix_v)
    pltpu.sync_copy(valsf.at[pl.ds(row0, 16)], vals_v)
    pltpu.sync_copy(clsf.at[pl.ds(row0, 16)], cls_v)

    pix = pix_v[...]
    lane = lax.bitwise_and(pix, 127)
    wordrow = lax.shift_right_logical(pix, 7)
    iota = lax.iota(jnp.int32, 16)

    ridx0_v[...] = b * 256 + wordrow
    ridx1_v[...] = b * 256 + 128 + wordrow
    handles_rw = [
        pltpu.async_copy(reg_t.at[ridx0_v], rg_rows.at[0], sem_rw),
        pltpu.async_copy(reg_t.at[ridx1_v], rg_rows.at[1], sem_rw),
        pltpu.async_copy(wh_t.at[ridx0_v], rg_rows.at[2], sem_rw),
        pltpu.async_copy(wh_t.at[ridx1_v], rg_rows.at[3], sem_rw),
    ]

    for k in range(16):
        pk = jnp.max(jnp.where(iota == k, pix, -1))
        rowbase = b * _CWROWS + lax.shift_right_logical(pk, 7)
        for j2 in range(2):
            for t in range(6):
                ch = jnp.minimum(iota + (j2 * 96 + t * 16), 168)
                plsc.store_scatter(idx_v, [iota + (k * 192 + j2 * 96 + t * 16)],
                                   rowbase + ch * 128)

    def fire(pos):
        slot = pos % 4
        k, j2 = pos // 2, pos % 2
        return pltpu.async_copy(
            cw_t.at[idx_v.at[pl.ds(k * 192 + j2 * 96, 96)]],
            rbuf.at[slot], slot_sems[slot])

    handles = {}
    for pos in range(4):
        handles[pos] = fire(pos)
    for pos in range(32):
        k, j2 = pos // 2, pos % 2
        slot = pos % 4
        handles.pop(pos).wait()
        pk = jnp.max(jnp.where(iota == k, pix, -1))
        lk = jnp.full((16,), lax.bitwise_and(pk, 127), jnp.int32)
        for jj in range(6 if j2 == 0 else 5):
            cbase = j2 * 96 + jj * 16
            vals = plsc.load_gather(rbuf.at[slot], [iota + jj * 16, lk])
            plsc.store_scatter(conv_res, [iota + (k * _NCV + cbase)], vals)
        if pos + 4 < 32:
            handles[pos + 4] = fire(pos + 4)

    for h in handles_rw:
        h.wait()
    r0 = plsc.load_gather(rg_rows, [jnp.full((16,), 0, jnp.int32), iota, lane])
    r1 = plsc.load_gather(rg_rows, [jnp.full((16,), 1, jnp.int32), iota, lane])
    w0 = plsc.load_gather(rg_rows, [jnp.full((16,), 2, jnp.int32), iota, lane])
    w1 = plsc.load_gather(rg_rows, [jnp.full((16,), 3, jnp.int32), iota, lane])

    xs = lax.bitwise_and(pix, _W - 1).astype(jnp.float32)
    ys = lax.shift_right_logical(pix, 7).astype(jnp.float32)
    cx = xs + r0
    cy = ys + r1
    cols = [cx - w0 / 2, cy - w1 / 2, cx + w0 / 2, cy + w1 / 2,
            vals_v[...], cls_v[...].astype(jnp.float32),
            jnp.zeros((16,), jnp.float32), jnp.zeros((16,), jnp.float32)]
    for j, colv in enumerate(cols):
        plsc.store_scatter(bbox_v, [iota * 8 + j], colv)

    pltpu.sync_copy(bbox_v, bbox_out.at[pl.ds(row0 * 8, 128)])
    pltpu.sync_copy(conv_res, convw_out.at[pl.ds(row0 * _NCV, 16 * _NCV)])


def _run_sc_gather(reg_t, wh_t, cw_t, pixf, valsf, clsf):
    return pl.kernel(
        _sc_gather_body,
        mesh=plsc.VectorSubcoreMesh(core_axis_name="c", subcore_axis_name="s"),
        compiler_params=pltpu.CompilerParams(needs_layout_passes=False),
        out_type=[
            jax.ShapeDtypeStruct((_B * _K * 8,), jnp.float32),
            jax.ShapeDtypeStruct((_B * _K * _NCV,), jnp.float32),
        ],
        scratch_types=[
            pltpu.VMEM((16,), jnp.int32),
            pltpu.VMEM((16,), jnp.float32),
            pltpu.VMEM((16,), jnp.int32),
            pltpu.VMEM((16,), jnp.int32),
            pltpu.VMEM((16,), jnp.int32),
            pltpu.VMEM((16 * 192,), jnp.int32),
            pltpu.VMEM((4, 96, 128), jnp.float32),
            pltpu.VMEM((4, 16, 128), jnp.float32),
            pltpu.VMEM((16 * _NCV,), jnp.float32),
            pltpu.VMEM((128,), jnp.float32),
            pltpu.SemaphoreType.DMA,
            pltpu.SemaphoreType.DMA,
            pltpu.SemaphoreType.DMA,
            pltpu.SemaphoreType.DMA,
            pltpu.SemaphoreType.DMA,
        ],
    )(reg_t, wh_t, cw_t, pixf, valsf, clsf)


def kernel(hm, reg, wh, seg_feat, conv_weight):
    vals3, pix3, cls3 = _run_topk(hm)
    bbox_flat, convw = _run_sc_gather(
        reg.reshape(_B * 2 * _H, _W),
        wh.reshape(_B * 2 * _H, _W),
        conv_weight.reshape(_B * 169 * _H, _W),
        pix3.reshape(_B * _K),
        vals3.reshape(_B * _K),
        cls3.reshape(_B * _K),
    )
    bboxes = bbox_flat.reshape(_B, _K, 8)[:, :, :6]
    conv_g = convw.reshape(_B, _K, _NCV)[:, :, :169]
    return (bboxes, seg_feat, conv_g)


# speedup vs baseline: 1.5334x; 1.1190x over previous
import functools

import jax
import jax.numpy as jnp
from jax import lax
from jax.experimental import pallas as pl
from jax.experimental.pallas import tpu as pltpu
from jax.experimental.pallas import tpu_sc as plsc

_B, _C, _H, _W = 8, 80, 128, 128
_K = 64
_CBLK = 16
_NCHUNK = _C // _CBLK
_BIG = 2**30


_BG = 4


def _topk_body(hm_ref, vals_ref, pix_ref, cls_ref, *scrs):
    nms_scrs = scrs[:_BG]
    sum_scrs = scrs[_BG:2 * _BG]
    cc = pl.program_id(1)
    s = jax.nn.sigmoid(hm_ref[...])
    zw = jnp.zeros((_BG, _CBLK, _H, 1), jnp.float32)
    m3 = jnp.maximum(s, jnp.maximum(
        jnp.concatenate([zw, s[:, :, :, :-1]], axis=3),
        jnp.concatenate([s[:, :, :, 1:], zw], axis=3)))
    zh = jnp.zeros((_BG, _CBLK, 1, _W), jnp.float32)
    hmax = jnp.maximum(m3, jnp.maximum(
        jnp.concatenate([zh, m3[:, :, :-1, :]], axis=2),
        jnp.concatenate([m3[:, :, 1:, :], zh], axis=2)))
    nms = jnp.where(s == hmax, s, 0.0)
    rowmax = jnp.max(nms, axis=3)
    for g in range(_BG):
        nms_scrs[g][pl.ds(cc * _CBLK, _CBLK)] = nms[g]
        sum_scrs[g][pl.ds(cc * _CBLK, _CBLK)] = rowmax[g]

    @pl.when(cc == _NCHUNK - 1)
    def _extract():
        ch_iota = (lax.broadcasted_iota(jnp.int32, (_C, _H), 0) * _H
                   + lax.broadcasted_iota(jnp.int32, (_C, _H), 1))
        w_iota = lax.broadcasted_iota(jnp.int32, (1, 1, _W), 2)
        h_iota = lax.broadcasted_iota(jnp.int32, (1, _H), 1)

        def body(k, carry):
            for g in range(_BG):
                nms_scr, sum_scr = nms_scrs[g], sum_scrs[g]
                summ = sum_scr[...]
                m = jnp.max(summ)
                f = jnp.min(jnp.where(summ == m, ch_iota, _BIG))
                c = f // _H
                h = f - c * _H
                row = nms_scr[pl.ds(c, 1), pl.ds(h, 1), :]
                w = jnp.min(jnp.where(row == m, w_iota, _BIG))
                vals_ref[g, pl.ds(k, 1), :] = jnp.full((1, 1), m, jnp.float32)
                pix_ref[g, pl.ds(k, 1), :] = jnp.full((1, 1), h * _W + w,
                                                      jnp.int32)
                cls_ref[g, pl.ds(k, 1), :] = jnp.full((1, 1), c, jnp.int32)
                row2 = jnp.where(w_iota == w, 0.0, row)
                nms_scr[pl.ds(c, 1), pl.ds(h, 1), :] = row2
                srow = sum_scr[pl.ds(c, 1), :]
                sum_scr[pl.ds(c, 1), :] = jnp.where(h_iota == h,
                                                    jnp.max(row2), srow)
            return carry

        lax.fori_loop(0, _K, body, 0)


def _run_topk(hm, interpret=False):
    return pl.pallas_call(
        _topk_body,
        grid=(_B // _BG, _NCHUNK),
        in_specs=[pl.BlockSpec((_BG, _CBLK, _H, _W),
                               lambda g, cc: (g, cc, 0, 0))],
        out_specs=[
            pl.BlockSpec((_BG, _K, 1), lambda g, cc: (g, 0, 0)),
            pl.BlockSpec((_BG, _K, 1), lambda g, cc: (g, 0, 0)),
            pl.BlockSpec((_BG, _K, 1), lambda g, cc: (g, 0, 0)),
        ],
        out_shape=[
            jax.ShapeDtypeStruct((_B, _K, 1), jnp.float32),
            jax.ShapeDtypeStruct((_B, _K, 1), jnp.int32),
            jax.ShapeDtypeStruct((_B, _K, 1), jnp.int32),
        ],
        scratch_shapes=(
            [pltpu.VMEM((_C, _H, _W), jnp.float32) for _ in range(_BG)]
            + [pltpu.VMEM((_C, _H), jnp.float32) for _ in range(_BG)]),
        interpret=interpret,
    )(hm)


_NCV = 176
_CWROWS = 169 * 128


def _sc_gather_body(reg_t, wh_t, cw_t, pixf, valsf, clsf,
                    bbox_out, convw_out,
                    pix_v, vals_v, cls_v, ridx0_v, ridx1_v, idx_v, rbuf,
                    rg_rows, conv_res, bbox_v, sem_rw, s0, s1, s2, s3):
    slot_sems = (s0, s1, s2, s3)
    nc = 2
    wid = lax.axis_index("s") * nc + lax.axis_index("c")
    b = wid // 4
    row0 = b * _K + (wid % 4) * 16

    pltpu.sync_copy(pixf.at[pl.ds(row0, 16)], pix_v)
    pltpu.sync_copy(valsf.at[pl.ds(row0, 16)], vals_v)
    pltpu.sync_copy(clsf.at[pl.ds(row0, 16)], cls_v)

    pix = pix_v[...]
    lane = lax.bitwise_and(pix, 127)
    wordrow = lax.shift_right_logical(pix, 7)
    iota = lax.iota(jnp.int32, 16)

    ridx0_v[...] = b * 256 + wordrow
    ridx1_v[...] = b * 256 + 128 + wordrow
    handles_rw = [
        pltpu.async_copy(reg_t.at[ridx0_v], rg_rows.at[0], sem_rw),
        pltpu.async_copy(reg_t.at[ridx1_v], rg_rows.at[1], sem_rw),
        pltpu.async_copy(wh_t.at[ridx0_v], rg_rows.at[2], sem_rw),
        pltpu.async_copy(wh_t.at[ridx1_v], rg_rows.at[3], sem_rw),
    ]

    for k in range(16):
        pk = jnp.max(jnp.where(iota == k, pix, -1))
        rowbase = b * _CWROWS + lax.shift_right_logical(pk, 7)
        for j2 in range(2):
            for t in range(6):
                ch = jnp.minimum(iota + (j2 * 96 + t * 16), 168)
                plsc.store_scatter(idx_v, [iota + (k * 192 + j2 * 96 + t * 16)],
                                   rowbase + ch * 128)

    def fire(pos):
        slot = pos % 4
        k, j2 = pos // 2, pos % 2
        return pltpu.async_copy(
            cw_t.at[idx_v.at[pl.ds(k * 192 + j2 * 96, 96)]],
            rbuf.at[slot], slot_sems[slot])

    handles = {}
    for pos in range(4):
        handles[pos] = fire(pos)
    for pos in range(32):
        k, j2 = pos // 2, pos % 2
        slot = pos % 4
        handles.pop(pos).wait()
        pk = jnp.max(jnp.where(iota == k, pix, -1))
        lk = jnp.full((16,), lax.bitwise_and(pk, 127), jnp.int32)
        for jj in range(6 if j2 == 0 else 5):
            cbase = j2 * 96 + jj * 16
            vals = plsc.load_gather(rbuf.at[slot], [iota + jj * 16, lk])
            plsc.store_scatter(conv_res, [iota + (k * _NCV + cbase)], vals)
        if pos + 4 < 32:
            handles[pos + 4] = fire(pos + 4)

    for h in handles_rw:
        h.wait()
    r0 = plsc.load_gather(rg_rows, [jnp.full((16,), 0, jnp.int32), iota, lane])
    r1 = plsc.load_gather(rg_rows, [jnp.full((16,), 1, jnp.int32), iota, lane])
    w0 = plsc.load_gather(rg_rows, [jnp.full((16,), 2, jnp.int32), iota, lane])
    w1 = plsc.load_gather(rg_rows, [jnp.full((16,), 3, jnp.int32), iota, lane])

    xs = lax.bitwise_and(pix, _W - 1).astype(jnp.float32)
    ys = lax.shift_right_logical(pix, 7).astype(jnp.float32)
    cx = xs + r0
    cy = ys + r1
    cols = [cx - w0 / 2, cy - w1 / 2, cx + w0 / 2, cy + w1 / 2,
            vals_v[...], cls_v[...].astype(jnp.float32),
            jnp.zeros((16,), jnp.float32), jnp.zeros((16,), jnp.float32)]
    for j, colv in enumerate(cols):
        plsc.store_scatter(bbox_v, [iota * 8 + j], colv)

    pltpu.sync_copy(bbox_v, bbox_out.at[pl.ds(row0 * 8, 128)])
    pltpu.sync_copy(conv_res, convw_out.at[pl.ds(row0 * _NCV, 16 * _NCV)])


def _run_sc_gather(reg_t, wh_t, cw_t, pixf, valsf, clsf):
    return pl.kernel(
        _sc_gather_body,
        mesh=plsc.VectorSubcoreMesh(core_axis_name="c", subcore_axis_name="s"),
        compiler_params=pltpu.CompilerParams(needs_layout_passes=False),
        out_type=[
            jax.ShapeDtypeStruct((_B * _K * 8,), jnp.float32),
            jax.ShapeDtypeStruct((_B * _K * _NCV,), jnp.float32),
        ],
        scratch_types=[
            pltpu.VMEM((16,), jnp.int32),
            pltpu.VMEM((16,), jnp.float32),
            pltpu.VMEM((16,), jnp.int32),
            pltpu.VMEM((16,), jnp.int32),
            pltpu.VMEM((16,), jnp.int32),
            pltpu.VMEM((16 * 192,), jnp.int32),
            pltpu.VMEM((4, 96, 128), jnp.float32),
            pltpu.VMEM((4, 16, 128), jnp.float32),
            pltpu.VMEM((16 * _NCV,), jnp.float32),
            pltpu.VMEM((128,), jnp.float32),
            pltpu.SemaphoreType.DMA,
            pltpu.SemaphoreType.DMA,
            pltpu.SemaphoreType.DMA,
            pltpu.SemaphoreType.DMA,
            pltpu.SemaphoreType.DMA,
        ],
    )(reg_t, wh_t, cw_t, pixf, valsf, clsf)


def kernel(hm, reg, wh, seg_feat, conv_weight):
    vals3, pix3, cls3 = _run_topk(hm)
    bbox_flat, convw = _run_sc_gather(
        reg.reshape(_B * 2 * _H, _W),
        wh.reshape(_B * 2 * _H, _W),
        conv_weight.reshape(_B * 169 * _H, _W),
        pix3.reshape(_B * _K),
        vals3.reshape(_B * _K),
        cls3.reshape(_B * _K),
    )
    bboxes = bbox_flat.reshape(_B, _K, 8)[:, :, :6]
    conv_g = convw.reshape(_B, _K, _NCV)[:, :, :169]
    return (bboxes, seg_feat, conv_g)
